# Initial kernel scaffold; baseline (speedup 1.0000x reference)
import functools

import jax
import jax.numpy as jnp
from jax import lax
from jax.experimental import pallas as pl
from jax.experimental.pallas import tpu as pltpu
from jax.experimental.pallas import tpu_sc as plsc

NC = 2
NS = 16
NW = NC * NS
BLK = 128


def _pad_to(arr, n, fill):
    pad = n - arr.shape[0]
    if pad == 0:
        return arr
    return jnp.concatenate([arr, jnp.full((pad,), fill, arr.dtype)], axis=0)


def _sc_gather(table, idx):
    V, D = table.shape
    B = idx.shape[0]
    per_w = B // NW
    nblk = per_w // BLK
    mesh = plsc.VectorSubcoreMesh(core_axis_name="c", subcore_axis_name="s")

    @functools.partial(
        pl.kernel,
        mesh=mesh,
        out_type=jax.ShapeDtypeStruct((B, D), table.dtype),
        scratch_types=[
            pltpu.VMEM((BLK,), jnp.int32),
            pltpu.VMEM((BLK, D), table.dtype),
            pltpu.SemaphoreType.DMA,
        ],
    )
    def k(table_hbm, idx_hbm, out_hbm, idx_v, rows_v, sem):
        wid = lax.axis_index("s") * NC + lax.axis_index("c")
        base = wid * per_w

        @pl.loop(0, nblk)
        def _(i):
            off = base + i * BLK
            pltpu.sync_copy(idx_hbm.at[pl.ds(off, BLK)], idx_v)
            pltpu.async_copy(table_hbm.at[idx_v], rows_v, sem).wait()
            pltpu.sync_copy(rows_v, out_hbm.at[pl.ds(off, BLK), :])

    return k(table, idx)


def _sc_segment_sum(vals, dst, n_nodes):
    E, D = vals.shape
    per_w = E // NW
    nblk = per_w // BLK
    rows_per_sub = n_nodes // NS
    zchunk = 125
    mesh = plsc.VectorSubcoreMesh(core_axis_name="c", subcore_axis_name="s")

    @functools.partial(
        pl.kernel,
        mesh=mesh,
        out_type=jax.ShapeDtypeStruct((NC, n_nodes, D), vals.dtype),
        scratch_types=[
            pltpu.VMEM((BLK,), jnp.int32),
            pltpu.VMEM((BLK, D), vals.dtype),
            pltpu.VMEM_SHARED((n_nodes, D), vals.dtype),
        ],
    )
    def k(vals_hbm, dst_hbm, out_hbm, idx_v, rows_v, agg_sh):
        cid = lax.axis_index("c")
        sid = lax.axis_index("s")
        wid = sid * NC + cid

        @pl.loop(0, BLK)
        def _(i):
            @pl.loop(0, D // 16)
            def _(j):
                rows_v[i, pl.ds(j * 16, 16)] = jnp.zeros((16,), vals.dtype)

        @pl.loop(0, rows_per_sub // zchunk)
        def _(i):
            r0 = sid * rows_per_sub + i * zchunk
            pltpu.sync_copy(rows_v.at[pl.ds(0, zchunk), :],
                            agg_sh.at[pl.ds(r0, zchunk), :])

        plsc.subcore_barrier()

        base = wid * per_w

        @pl.loop(0, nblk)
        def _(i):
            off = base + i * BLK
            pltpu.sync_copy(dst_hbm.at[pl.ds(off, BLK)], idx_v)
            pltpu.sync_copy(vals_hbm.at[pl.ds(off, BLK), :], rows_v)
            pltpu.sync_copy(rows_v, agg_sh.at[idx_v], add=True)

        plsc.subcore_barrier()

        r0 = sid * rows_per_sub
        pltpu.sync_copy(agg_sh.at[pl.ds(r0, rows_per_sub), :],
                        out_hbm.at[cid, pl.ds(r0, rows_per_sub), :])

    return k(vals, dst)


def _tc_scale(g, w_col):
    E, D = g.shape
    blk = 2048

    def body(g_ref, w_ref, o_ref):
        o_ref[...] = g_ref[...] * w_ref[...]

    return pl.pallas_call(
        body,
        grid=(E // blk,),
        in_specs=[pl.BlockSpec((blk, D), lambda i: (i, 0)),
                  pl.BlockSpec((blk, 1), lambda i: (i, 0))],
        out_specs=pl.BlockSpec((blk, D), lambda i: (i, 0)),
        out_shape=jax.ShapeDtypeStruct((E, D), g.dtype),
    )(g, w_col)


def _tc_linear(partials, W, b_row):
    _, N, D = partials.shape
    blk = 2000

    def body(p_ref, w_ref, b_ref, o_ref):
        s = p_ref[0] + p_ref[1]
        o_ref[...] = jnp.dot(s, w_ref[...],
                             preferred_element_type=jnp.float32) + b_ref[...]

    return pl.pallas_call(
        body,
        grid=(N // blk,),
        in_specs=[pl.BlockSpec((NC, blk, D), lambda i: (0, i, 0)),
                  pl.BlockSpec((D, D), lambda i: (0, 0)),
                  pl.BlockSpec((1, D), lambda i: (0, 0))],
        out_specs=pl.BlockSpec((blk, D), lambda i: (i, 0)),
        out_shape=jax.ShapeDtypeStruct((N, D), jnp.float32),
    )(partials, W, b_row)


def _tc_rowdot(ha, hb):
    C, D = ha.shape
    blk = 2048

    def body(a_ref, b_ref, o_ref):
        prod = a_ref[...] * b_ref[...]
        ones = jnp.ones((D, 1), jnp.float32)
        o_ref[...] = jnp.dot(prod, ones, preferred_element_type=jnp.float32)

    return pl.pallas_call(
        body,
        grid=(C // blk,),
        in_specs=[pl.BlockSpec((blk, D), lambda i: (i, 0)),
                  pl.BlockSpec((blk, D), lambda i: (i, 0))],
        out_specs=pl.BlockSpec((blk, 1), lambda i: (i, 0)),
        out_shape=jax.ShapeDtypeStruct((C, 1), jnp.float32),
    )(ha, hb)


def kernel(x, edge_index, edge_weight, edges, W, b):
    n_nodes, d = x.shape
    n_edges = edge_weight.shape[0]
    n_cand = edges.shape[1]

    unit = NW * BLK
    e_pad = ((n_edges + unit - 1) // unit) * unit
    c_pad = ((n_cand + unit - 1) // unit) * unit

    src = _pad_to(edge_index[0].astype(jnp.int32), e_pad, 0)
    dst = _pad_to(edge_index[1].astype(jnp.int32), e_pad, 0)
    w_col = _pad_to(edge_weight, e_pad, 0.0).reshape(e_pad, 1)
    e0 = _pad_to(edges[0].astype(jnp.int32), c_pad, 0)
    e1 = _pad_to(edges[1].astype(jnp.int32), c_pad, 0)

    g = _sc_gather(x, src)
    gw = _tc_scale(g, w_col)
    partials = _sc_segment_sum(gw, dst, n_nodes)
    h = _tc_linear(partials, W, b.reshape(1, d))
    ha = _sc_gather(h, e0)
    hb = _sc_gather(h, e1)
    out = _tc_rowdot(ha, hb)
    return out[:n_cand, 0]


# speedup vs baseline: 2.0696x; 2.0696x over previous
import functools

import jax
import jax.numpy as jnp
from jax import lax
from jax.experimental import pallas as pl
from jax.experimental.pallas import tpu as pltpu
from jax.experimental.pallas import tpu_sc as plsc

NC = 2
NS = 16
NW = NC * NS
BLK = 128


def _pad_to(arr, n, fill):
    pad = n - arr.shape[0]
    if pad == 0:
        return arr
    return jnp.concatenate([arr, jnp.full((pad,), fill, arr.dtype)], axis=0)


def _sc_gather(table, idx):
    V, D = table.shape
    B = idx.shape[0]
    per_w = B // NW
    nblk = per_w // BLK
    mesh = plsc.VectorSubcoreMesh(core_axis_name="c", subcore_axis_name="s")

    @functools.partial(
        pl.kernel,
        mesh=mesh,
        out_type=jax.ShapeDtypeStruct((B, D), table.dtype),
        scratch_types=[
            pltpu.VMEM((BLK,), jnp.int32),
            pltpu.VMEM((BLK, D), table.dtype),
            pltpu.SemaphoreType.DMA,
        ],
    )
    def k(table_hbm, idx_hbm, out_hbm, idx_v, rows_v, sem):
        wid = lax.axis_index("s") * NC + lax.axis_index("c")
        base = wid * per_w

        @pl.loop(0, nblk)
        def _(i):
            off = base + i * BLK
            pltpu.sync_copy(idx_hbm.at[pl.ds(off, BLK)], idx_v)
            pltpu.async_copy(table_hbm.at[idx_v], rows_v, sem).wait()
            pltpu.sync_copy(rows_v, out_hbm.at[pl.ds(off, BLK), :])

    return k(table, idx)


def _sc_segment_sum(vals, dst, n_nodes):
    E, D = vals.shape
    per_w = E // NW
    nblk = per_w // BLK
    rows_per_sub = n_nodes // NS
    zchunk = BLK
    mesh = plsc.VectorSubcoreMesh(core_axis_name="c", subcore_axis_name="s")

    @functools.partial(
        pl.kernel,
        mesh=mesh,
        out_type=jax.ShapeDtypeStruct((NC, n_nodes, D), vals.dtype),
        scratch_types=[
            pltpu.VMEM((BLK,), jnp.int32),
            pltpu.VMEM((BLK, D), vals.dtype),
            pltpu.VMEM_SHARED((n_nodes, D), vals.dtype),
        ],
    )
    def k(vals_hbm, dst_hbm, out_hbm, idx_v, rows_v, agg_sh):
        cid = lax.axis_index("c")
        sid = lax.axis_index("s")
        wid = sid * NC + cid

        @pl.loop(0, BLK)
        def _(i):
            @pl.loop(0, D // 16)
            def _(j):
                rows_v[i, pl.ds(j * 16, 16)] = jnp.zeros((16,), vals.dtype)

        @pl.loop(0, rows_per_sub // zchunk)
        def _(i):
            r0 = sid * rows_per_sub + i * zchunk
            pltpu.sync_copy(rows_v.at[pl.ds(0, zchunk), :],
                            agg_sh.at[pl.ds(r0, zchunk), :])

        plsc.subcore_barrier()

        base = wid * per_w

        @pl.loop(0, nblk)
        def _(i):
            off = base + i * BLK
            pltpu.sync_copy(dst_hbm.at[pl.ds(off, BLK)], idx_v)
            pltpu.sync_copy(vals_hbm.at[pl.ds(off, BLK), :], rows_v)
            pltpu.sync_copy(rows_v, agg_sh.at[idx_v], add=True)

        plsc.subcore_barrier()

        r0 = sid * rows_per_sub
        pltpu.sync_copy(agg_sh.at[pl.ds(r0, rows_per_sub), :],
                        out_hbm.at[cid, pl.ds(r0, rows_per_sub), :])

    return k(vals, dst)


def _tc_scale(g, w_col):
    E, D = g.shape
    blk = 2048

    def body(g_ref, w_ref, o_ref):
        o_ref[...] = g_ref[...] * w_ref[...]

    return pl.pallas_call(
        body,
        grid=(E // blk,),
        in_specs=[pl.BlockSpec((blk, D), lambda i: (i, 0)),
                  pl.BlockSpec((blk, 1), lambda i: (i, 0))],
        out_specs=pl.BlockSpec((blk, D), lambda i: (i, 0)),
        out_shape=jax.ShapeDtypeStruct((E, D), g.dtype),
    )(g, w_col)


def _tc_linear(partials, W, b_row):
    _, N, D = partials.shape
    blk = 2048

    def body(p_ref, w_ref, b_ref, o_ref):
        s = p_ref[0] + p_ref[1]
        o_ref[...] = jnp.dot(s, w_ref[...],
                             preferred_element_type=jnp.float32) + b_ref[...]

    return pl.pallas_call(
        body,
        grid=(N // blk,),
        in_specs=[pl.BlockSpec((NC, blk, D), lambda i: (0, i, 0)),
                  pl.BlockSpec((D, D), lambda i: (0, 0)),
                  pl.BlockSpec((1, D), lambda i: (0, 0))],
        out_specs=pl.BlockSpec((blk, D), lambda i: (i, 0)),
        out_shape=jax.ShapeDtypeStruct((N, D), jnp.float32),
    )(partials, W, b_row)


def _tc_rowdot(ha, hb):
    C, D = ha.shape
    blk = 2048

    def body(a_ref, b_ref, o_ref):
        prod = a_ref[...] * b_ref[...]
        ones = jnp.ones((D, 1), jnp.float32)
        o_ref[...] = jnp.dot(prod, ones, preferred_element_type=jnp.float32)

    return pl.pallas_call(
        body,
        grid=(C // blk,),
        in_specs=[pl.BlockSpec((blk, D), lambda i: (i, 0)),
                  pl.BlockSpec((blk, D), lambda i: (i, 0))],
        out_specs=pl.BlockSpec((blk, 1), lambda i: (i, 0)),
        out_shape=jax.ShapeDtypeStruct((C, 1), jnp.float32),
    )(ha, hb)


def kernel(x, edge_index, edge_weight, edges, W, b):
    n_nodes, d = x.shape
    n_edges = edge_weight.shape[0]
    n_cand = edges.shape[1]

    unit = NW * BLK
    e_pad = ((n_edges + unit - 1) // unit) * unit
    c_pad = ((n_cand + unit - 1) // unit) * unit

    src = _pad_to(edge_index[0].astype(jnp.int32), e_pad, 0)
    dst = _pad_to(edge_index[1].astype(jnp.int32), e_pad, 0)
    w_col = _pad_to(edge_weight, e_pad, 0.0).reshape(e_pad, 1)
    e0 = _pad_to(edges[0].astype(jnp.int32), c_pad, 0)
    e1 = _pad_to(edges[1].astype(jnp.int32), c_pad, 0)

    n_pad = ((n_nodes + NS * 8 - 1) // (NS * 8)) * (NS * 8)
    n_pad = max(n_pad, ((n_nodes + 2047) // 2048) * 2048)

    g = _sc_gather(x, src)
    gw = _tc_scale(g, w_col)
    partials = _sc_segment_sum(gw, dst, n_pad)
    h = _tc_linear(partials, W, b.reshape(1, d))
    ha = _sc_gather(h, e0)
    hb = _sc_gather(h, e1)
    out = _tc_rowdot(ha, hb)
    return out[:n_cand, 0]
